# baseline (speedup 1.0000x reference)
import functools

import jax
import jax.numpy as jnp
from jax.experimental import pallas as pl


def _mm_block(a_ref, b_ref, bias_ref, o_ref, *, relu):
    acc = jnp.dot(a_ref[...], b_ref[...], preferred_element_type=jnp.float32)
    acc = acc + bias_ref[...][None, :]
    if relu:
        acc = jnp.maximum(acc, 0.0)
    o_ref[...] = acc


def _mm(a, w, b, relu, block_n=512):
    m, k = a.shape
    n = w.shape[1]
    bn = min(block_n, n)
    assert n % bn == 0
    return pl.pallas_call(
        functools.partial(_mm_block, relu=relu),
        grid=(n // bn,),
        in_specs=[
            pl.BlockSpec((m, k), lambda j: (0, 0)),
            pl.BlockSpec((k, bn), lambda j: (0, j)),
            pl.BlockSpec((bn,), lambda j: (j,)),
        ],
        out_specs=pl.BlockSpec((m, bn), lambda j: (0, j)),
        out_shape=jax.ShapeDtypeStruct((m, n), jnp.float32),
    )(a, w, b)


def _bn(h, p):
    mean = jnp.mean(h, axis=0)
    var = jnp.var(h, axis=0)
    return (h - mean) / jnp.sqrt(var + 1e-5) * p["gamma"] + p["beta"]


def _gcn(h, src, dst, p, n):
    h = h @ p["W"]
    deg = jax.ops.segment_sum(jnp.ones(src.shape[0], dtype=h.dtype), dst, num_segments=n)
    dinv = jnp.where(deg > 0, 1.0 / jnp.sqrt(deg), 0.0)
    norm = dinv[src] * dinv[dst]
    return jax.ops.segment_sum(h[src] * norm[:, None], dst, num_segments=n) + p["b"]


def _gat(h, src, dst, p, n):
    h = h @ p["W"]
    a_s = h @ p["att_src"]
    a_d = h @ p["att_dst"]
    e = jax.nn.leaky_relu(a_s[src] + a_d[dst], 0.2)
    m = jax.ops.segment_max(e, dst, num_segments=n)
    m = jnp.where(jnp.isfinite(m), m, 0.0)
    ex = jnp.exp(e - m[dst])
    denom = jax.ops.segment_sum(ex, dst, num_segments=n)
    alpha = ex / (denom[dst] + 1e-16)
    return jax.ops.segment_sum(alpha[:, None] * h[src], dst, num_segments=n) + p["b"]


def kernel(x, ei, batch, params):
    n = x.shape[0]
    loops = jnp.arange(n, dtype=ei.dtype)
    src = jnp.concatenate([ei[0], loops])
    dst = jnp.concatenate([ei[1], loops])
    h = x.astype(jnp.float32)
    bn_i = 0
    for p in params["gcn"]:
        h = _gcn(h, src, dst, p, n)
        h = jax.nn.relu(_bn(h, params["bn"][bn_i])); bn_i += 1
    for p in params["gat"]:
        h = _gat(h, src, dst, p, n)
        h = jax.nn.relu(_bn(h, params["bn"][bn_i])); bn_i += 1
    g = jax.ops.segment_max(h, batch, num_segments=64)
    g = jnp.where(jnp.isfinite(g), g, 0.0)
    g = _mm(g, params["fc"][0]["W"], params["fc"][0]["b"], relu=True)
    g = _mm(g, params["fc"][1]["W"], params["fc"][1]["b"], relu=True)
    w3 = jnp.pad(params["fc"][2]["W"], ((0, 0), (0, 126)))
    b3 = jnp.pad(params["fc"][2]["b"], (0, 126))
    out = _mm(g, w3, b3, relu=False, block_n=128)
    return out[:, :2]
